# Initial kernel scaffold; baseline (speedup 1.0000x reference)
import functools

import jax
import jax.numpy as jnp
from jax import lax
from jax.experimental import pallas as pl
from jax.experimental.pallas import tpu as pltpu
from jax.experimental.pallas import tpu_sc as plsc

N = 10000
E = 320000
H = 128
OUT_DIM = 64
BETA = 0.5
CVAL = 1.0

NC = 2
NS = 16
LANES = 16
NBLK = H // LANES

CHUNK = 128
TILE_EDGES = E // (NC * NS)
FULL_CHUNKS = TILE_EDGES // CHUNK
REM = TILE_EDGES - FULL_CHUNKS * CHUNK
TILE_ROWS = N // NS
DRAIN = 125

_f32 = jnp.float32
_HIGH = lax.Precision.HIGHEST


def _sc_body(z_hbm, src_hbm, dst_hbm, w_hbm, out_hbm,
             srcv, dstv, wv, rows, srcr, dstr, wr, rowsr, acc, sem):
    cid = lax.axis_index("c")
    sid = lax.axis_index("s")

    @pl.loop(0, DRAIN)
    def _zero(i):
        for b in range(NBLK):
            rows[i, pl.ds(b * LANES, LANES)] = jnp.zeros((LANES,), _f32)

    row0 = sid * TILE_ROWS
    for r in range(TILE_ROWS // DRAIN):
        pltpu.sync_copy(rows.at[pl.ds(0, DRAIN)],
                        acc.at[pl.ds(row0 + r * DRAIN, DRAIN)])

    plsc.subcore_barrier()

    ebase = (cid * NS + sid) * TILE_EDGES

    def _scale(rows_ref, w_ref, count):
        @pl.loop(0, count)
        def _(e):
            wb = plsc.load_gather(w_ref, [jnp.full((LANES,), e, jnp.int32)])
            for b in range(NBLK):
                sl = pl.ds(b * LANES, LANES)
                rows_ref[e, sl] = rows_ref[e, sl] * wb

    @pl.loop(0, FULL_CHUNKS)
    def _edges(j):
        off = ebase + j * CHUNK
        pltpu.sync_copy(src_hbm.at[pl.ds(off, CHUNK)], srcv)
        pltpu.sync_copy(dst_hbm.at[pl.ds(off, CHUNK)], dstv)
        pltpu.sync_copy(w_hbm.at[pl.ds(off, CHUNK)], wv)
        pltpu.async_copy(z_hbm.at[srcv], rows, sem).wait()
        _scale(rows, wv, CHUNK)
        pltpu.sync_copy(rows, acc.at[dstv], add=True)

    offr = ebase + FULL_CHUNKS * CHUNK
    pltpu.sync_copy(src_hbm.at[pl.ds(offr, REM)], srcr)
    pltpu.sync_copy(dst_hbm.at[pl.ds(offr, REM)], dstr)
    pltpu.sync_copy(w_hbm.at[pl.ds(offr, REM)], wr)
    pltpu.async_copy(z_hbm.at[srcr], rowsr, sem).wait()
    _scale(rowsr, wr, REM)
    pltpu.sync_copy(rowsr, acc.at[dstr], add=True)

    plsc.subcore_barrier()

    for r in range(TILE_ROWS // DRAIN):
        sl = pl.ds(row0 + r * DRAIN, DRAIN)
        pltpu.sync_copy(acc.at[sl], out_hbm.at[cid].at[sl])


def _sc_agg(z, src, dst, w):
    mesh = plsc.VectorSubcoreMesh(core_axis_name="c", subcore_axis_name="s",
                                  num_cores=NC, num_subcores=NS)
    kfn = pl.kernel(
        _sc_body,
        out_type=jax.ShapeDtypeStruct((NC, N, H), _f32),
        mesh=mesh,
        scratch_types=[
            pltpu.VMEM((CHUNK,), jnp.int32),
            pltpu.VMEM((CHUNK,), jnp.int32),
            pltpu.VMEM((CHUNK,), _f32),
            pltpu.VMEM((CHUNK, H), _f32),
            pltpu.VMEM((REM,), jnp.int32),
            pltpu.VMEM((REM,), jnp.int32),
            pltpu.VMEM((REM,), _f32),
            pltpu.VMEM((REM, H), _f32),
            pltpu.VMEM_SHARED((N, H), _f32),
            pltpu.SemaphoreType.DMA,
        ],
    )
    return kfn(z, src, dst, w)


def _tc_first(x, W0):
    def body(x_ref, w_ref, o_ref):
        o_ref[...] = jnp.dot(x_ref[...], w_ref[...], precision=_HIGH,
                             preferred_element_type=_f32)
    return pl.pallas_call(
        body, out_shape=jax.ShapeDtypeStruct((N, H), _f32))(x, W0)


def _tc_mid(p, b, Wn):
    def body(p_ref, b_ref, w_ref, h_ref, z_ref):
        zagg = p_ref[0] + p_ref[1] + b_ref[...]
        h = BETA * zagg + (CVAL - BETA) * jnp.maximum(zagg, 0.0)
        h_ref[...] = h
        z_ref[...] = jnp.dot(h, w_ref[...], precision=_HIGH,
                             preferred_element_type=_f32)
    return pl.pallas_call(
        body,
        out_shape=(jax.ShapeDtypeStruct((N, H), _f32),
                   jax.ShapeDtypeStruct((N, H), _f32)),
    )(p, b.reshape(1, H), Wn)


def _tc_final(p, b2, h0, h1, Wlin, blin):
    def body(p_ref, b_ref, h0_ref, h1_ref, wl_ref, bl_ref, o_ref):
        zagg = p_ref[0] + p_ref[1] + b_ref[...]
        h2 = BETA * zagg + (CVAL - BETA) * jnp.maximum(zagg, 0.0)
        o_ref[...] = (
            jnp.dot(h0_ref[...], wl_ref[0:H], precision=_HIGH,
                    preferred_element_type=_f32)
            + jnp.dot(h1_ref[...], wl_ref[H:2 * H], precision=_HIGH,
                      preferred_element_type=_f32)
            + jnp.dot(h2, wl_ref[2 * H:3 * H], precision=_HIGH,
                      preferred_element_type=_f32)
            + bl_ref[...])
    return pl.pallas_call(
        body, out_shape=jax.ShapeDtypeStruct((N, OUT_DIM), _f32),
    )(p, b2.reshape(1, H), h0, h1, Wlin, blin)


def kernel(x, edge_index, edge_weight, W0, b0, W1, b1, W2, b2, Wlin, blin):
    src = edge_index[0]
    dst = edge_index[1]
    z = _tc_first(x, W0)
    p = _sc_agg(z, src, dst, edge_weight)
    h0, z = _tc_mid(p, b0, W1)
    p = _sc_agg(z, src, dst, edge_weight)
    h1, z = _tc_mid(p, b1, W2)
    p = _sc_agg(z, src, dst, edge_weight)
    return _tc_final(p, b2, h0, h1, Wlin, blin)


# speedup vs baseline: 4.3138x; 4.3138x over previous
import dataclasses
import functools

import jax
import jax.numpy as jnp
from jax import lax
from jax.experimental import pallas as pl
from jax.experimental.pallas import tpu as pltpu
from jax.experimental.pallas import tpu_sc as plsc

N = 10000
E = 320000
H = 128
OUT_DIM = 64
BETA = 0.5
CVAL = 1.0

NC = 2
NS = 16
LANES = 16
NBLK = H // LANES

CHUNK = 128
TILE_EDGES = E // (NC * NS)
FULL_CHUNKS = TILE_EDGES // CHUNK
REM = TILE_EDGES - FULL_CHUNKS * CHUNK
TILE_ROWS = 624
ROW_TAIL = N - NS * TILE_ROWS
ROW_CHUNKS = ((0, 128), (128, 128), (256, 128), (384, 128), (512, 112))

_f32 = jnp.float32
_HIGH = lax.Precision.HIGHEST


def _sc_body(z_hbm, src_hbm, dst_hbm, w_hbm, out_hbm,
             srcv, dstv, wv, rows, srcr, dstr, wr, rowsr, acc, sem):
    cid = lax.axis_index("c")
    sid = lax.axis_index("s")

    @pl.loop(0, CHUNK)
    def _zero(i):
        for b in range(NBLK):
            rows[i, pl.ds(b * LANES, LANES)] = jnp.zeros((LANES,), _f32)

    row0 = pl.multiple_of(sid * TILE_ROWS, 8)
    for off, sz in ROW_CHUNKS:
        pltpu.sync_copy(rows.at[pl.ds(0, sz)],
                        acc.at[pl.ds(row0 + off, sz)])

    @pl.when(sid == 0)
    def _zero_tail():
        pltpu.sync_copy(rows.at[pl.ds(0, ROW_TAIL)],
                        acc.at[pl.ds(NS * TILE_ROWS, ROW_TAIL)])

    plsc.subcore_barrier()

    ebase = pl.multiple_of((cid * NS + sid) * TILE_EDGES, 8)

    def _scale(rows_ref, w_ref, count):
        @pl.loop(0, count)
        def _(e):
            wb = plsc.load_gather(w_ref, [jnp.full((LANES,), e, jnp.int32)])
            for b in range(NBLK):
                sl = pl.ds(b * LANES, LANES)
                rows_ref[e, sl] = rows_ref[e, sl] * wb

    @pl.loop(0, FULL_CHUNKS)
    def _edges(j):
        off = ebase + j * CHUNK
        pltpu.sync_copy(src_hbm.at[pl.ds(off, CHUNK)], srcv)
        pltpu.sync_copy(dst_hbm.at[pl.ds(off, CHUNK)], dstv)
        pltpu.sync_copy(w_hbm.at[pl.ds(off, CHUNK)], wv)
        pltpu.async_copy(z_hbm.at[srcv], rows, sem).wait()
        _scale(rows, wv, CHUNK)
        pltpu.sync_copy(rows, acc.at[dstv], add=True)

    offr = ebase + FULL_CHUNKS * CHUNK
    pltpu.sync_copy(src_hbm.at[pl.ds(offr, REM)], srcr)
    pltpu.sync_copy(dst_hbm.at[pl.ds(offr, REM)], dstr)
    pltpu.sync_copy(w_hbm.at[pl.ds(offr, REM)], wr)
    pltpu.async_copy(z_hbm.at[srcr], rowsr, sem).wait()
    _scale(rowsr, wr, REM)
    pltpu.sync_copy(rowsr, acc.at[dstr], add=True)

    plsc.subcore_barrier()

    for off, sz in ROW_CHUNKS:
        sl = pl.ds(row0 + off, sz)
        pltpu.sync_copy(acc.at[sl], out_hbm.at[cid].at[sl])

    @pl.when(sid == 0)
    def _drain_tail():
        sl = pl.ds(NS * TILE_ROWS, ROW_TAIL)
        pltpu.sync_copy(acc.at[sl], out_hbm.at[cid].at[sl])


def _sc_params():
    cp = pltpu.CompilerParams()
    if "needs_layout_passes" in pltpu.CompilerParams.__dataclass_fields__:
        cp = dataclasses.replace(cp, needs_layout_passes=False)
    return cp


def _sc_agg(z, src, dst, w):
    mesh = plsc.VectorSubcoreMesh(core_axis_name="c", subcore_axis_name="s",
                                  num_cores=NC, num_subcores=NS)
    kfn = pl.kernel(
        _sc_body,
        out_type=jax.ShapeDtypeStruct((NC, N, H), _f32),
        mesh=mesh,
        scratch_types=[
            pltpu.VMEM((CHUNK,), jnp.int32),
            pltpu.VMEM((CHUNK,), jnp.int32),
            pltpu.VMEM((CHUNK,), _f32),
            pltpu.VMEM((CHUNK, H), _f32),
            pltpu.VMEM((REM,), jnp.int32),
            pltpu.VMEM((REM,), jnp.int32),
            pltpu.VMEM((REM,), _f32),
            pltpu.VMEM((REM, H), _f32),
            pltpu.VMEM_SHARED((N, H), _f32),
            pltpu.SemaphoreType.DMA,
        ],
        compiler_params=_sc_params(),
    )
    return kfn(z, src, dst, w)


BR = 1000
_GRID = N // BR


def _row_spec(width):
    return pl.BlockSpec((BR, width), lambda i: (i, 0))


def _full_spec(shape):
    return pl.BlockSpec(shape, lambda i: tuple(0 for _ in shape))


def _tc_first(x, W0):
    def body(x_ref, w_ref, o_ref):
        o_ref[...] = jnp.dot(x_ref[...], w_ref[...], precision=_HIGH,
                             preferred_element_type=_f32)
    return pl.pallas_call(
        body,
        grid=(_GRID,),
        in_specs=[_row_spec(H), _full_spec((H, H))],
        out_specs=_row_spec(H),
        out_shape=jax.ShapeDtypeStruct((N, H), _f32))(x, W0)


def _tc_mid(p, b, Wn):
    def body(p_ref, b_ref, w_ref, h_ref, z_ref):
        zagg = p_ref[0] + p_ref[1] + b_ref[...]
        h = BETA * zagg + (CVAL - BETA) * jnp.maximum(zagg, 0.0)
        h_ref[...] = h
        z_ref[...] = jnp.dot(h, w_ref[...], precision=_HIGH,
                             preferred_element_type=_f32)
    return pl.pallas_call(
        body,
        grid=(_GRID,),
        in_specs=[pl.BlockSpec((NC, BR, H), lambda i: (0, i, 0)),
                  _full_spec((1, H)), _full_spec((H, H))],
        out_specs=(_row_spec(H), _row_spec(H)),
        out_shape=(jax.ShapeDtypeStruct((N, H), _f32),
                   jax.ShapeDtypeStruct((N, H), _f32)),
    )(p, b.reshape(1, H), Wn)


def _tc_final(p, b2, h0, h1, Wlin, blin):
    def body(p_ref, b_ref, h0_ref, h1_ref, wl_ref, bl_ref, o_ref):
        zagg = p_ref[0] + p_ref[1] + b_ref[...]
        h2 = BETA * zagg + (CVAL - BETA) * jnp.maximum(zagg, 0.0)
        o_ref[...] = (
            jnp.dot(h0_ref[...], wl_ref[0:H], precision=_HIGH,
                    preferred_element_type=_f32)
            + jnp.dot(h1_ref[...], wl_ref[H:2 * H], precision=_HIGH,
                      preferred_element_type=_f32)
            + jnp.dot(h2, wl_ref[2 * H:3 * H], precision=_HIGH,
                      preferred_element_type=_f32)
            + bl_ref[...])
    return pl.pallas_call(
        body,
        grid=(_GRID,),
        in_specs=[pl.BlockSpec((NC, BR, H), lambda i: (0, i, 0)),
                  _full_spec((1, H)), _row_spec(H), _row_spec(H),
                  _full_spec((3 * H, OUT_DIM)), _full_spec((OUT_DIM,))],
        out_specs=_row_spec(OUT_DIM),
        out_shape=jax.ShapeDtypeStruct((N, OUT_DIM), _f32),
    )(p, b2.reshape(1, H), h0, h1, Wlin, blin)


def kernel(x, edge_index, edge_weight, W0, b0, W1, b1, W2, b2, Wlin, blin):
    src = edge_index[0]
    dst = edge_index[1]
    z = _tc_first(x, W0)
    p = _sc_agg(z, src, dst, edge_weight)
    h0, z = _tc_mid(p, b0, W1)
    p = _sc_agg(z, src, dst, edge_weight)
    h1, z = _tc_mid(p, b1, W2)
    p = _sc_agg(z, src, dst, edge_weight)
    return _tc_final(p, b2, h0, h1, Wlin, blin)
